# Initial kernel scaffold; baseline (speedup 1.0000x reference)
import jax
import jax.numpy as jnp
from jax.experimental import pallas as pl


def kernel(tokens, targets, tok_embed, enc_W0, enc_b0, enc_W1, enc_b1, enc_W2, enc_b2, enc_W3, enc_b3, codebook, dec_W0, dec_b0, dec_W1, dec_b1, dec_W2, dec_b2, dec_W3, dec_b3):
    raise NotImplementedError("write your pallas kernel here")


# speedup vs baseline: 1.1872x; 1.1872x over previous
import functools

import jax
import jax.numpy as jnp
from jax import lax
from jax.experimental import pallas as pl
from jax.experimental.pallas import tpu as pltpu
from jax.experimental.pallas import tpu_sc as plsc

B = 8192
NT = 32
TOK_EMB = 64
ED = 256
K = 8192
OUT = 2
BETA = 0.25

_BB1 = 256
_BB2 = 256
_NW = 32
_CHUNK = 128

_HI = lax.Precision.HIGHEST


def _mm(a, b):
    return lax.dot_general(a, b, (((1,), (0,)), ((), ())),
                           precision=_HI, preferred_element_type=jnp.float32)


def _sc_gather_rows(table, idx):
    n = idx.shape[0]
    d = table.shape[1]
    per_w = n // _NW
    n_chunks = per_w // _CHUNK
    mesh = plsc.VectorSubcoreMesh(core_axis_name="c", subcore_axis_name="s")

    @functools.partial(
        pl.kernel, mesh=mesh,
        compiler_params=pltpu.CompilerParams(use_tc_tiling_on_sc=False),
        out_type=jax.ShapeDtypeStruct((n, d), jnp.float32),
        scratch_types=[
            pltpu.VMEM((_CHUNK,), jnp.int32),
            pltpu.VMEM((_CHUNK, d), jnp.float32),
            pltpu.SemaphoreType.DMA,
        ],
    )
    def gather_k(idx_hbm, table_hbm, out_hbm, idx_v, rows_v, sem):
        wid = lax.axis_index("s") * 2 + lax.axis_index("c")
        base = wid * per_w

        def body(j, carry):
            off = base + j * _CHUNK
            pltpu.sync_copy(idx_hbm.at[pl.ds(off, _CHUNK)], idx_v)
            pltpu.async_copy(table_hbm.at[idx_v], rows_v, sem).wait()
            pltpu.sync_copy(rows_v, out_hbm.at[pl.ds(off, _CHUNK)])
            return carry

        lax.fori_loop(0, n_chunks, body, 0)

    return gather_k(idx, table)


def _enc_vq_body(e_ref, w0, b0, w1, b1, w2, b2, w3, b3, cb_ref,
                 ze_ref, idx_ref, c2_ref):
    @pl.when(pl.program_id(0) == 0)
    def _():
        c = cb_ref[...]
        c2_ref[...] = jnp.sum(c * c, axis=1)[None, :]

    h = e_ref[...]
    h = jnp.maximum(_mm(h, w0[...]) + b0[...], 0.0)
    h = jnp.maximum(_mm(h, w1[...]) + b1[...], 0.0)
    h = jnp.maximum(_mm(h, w2[...]) + b2[...], 0.0)
    z = _mm(h, w3[...]) + b3[...]
    ze_ref[...] = z

    s = lax.dot_general(z, cb_ref[...], (((1,), (1,)), ((), ())),
                        precision=_HI, preferred_element_type=jnp.float32)
    z2 = jnp.sum(z * z, axis=1, keepdims=True)
    dist = (z2 - 2.0 * s) + c2_ref[...]
    m = jnp.min(dist, axis=1, keepdims=True)
    ii = lax.broadcasted_iota(jnp.int32, dist.shape, 1)
    idx = jnp.min(jnp.where(dist == m, ii, jnp.int32(K)), axis=1)
    idx_ref[...] = idx[:, None].astype(jnp.int32)


def _enc_vq(e, ws, bs, cb):
    nb = B // _BB1
    full2 = lambda shape: pl.BlockSpec(shape, lambda i: (0, 0))
    in_specs = [pl.BlockSpec((_BB1, NT * TOK_EMB), lambda i: (i, 0))]
    for w, b in zip(ws, bs):
        in_specs.append(full2(w.shape))
        in_specs.append(full2(b.shape))
    in_specs.append(full2(cb.shape))
    flat = []
    for w, b in zip(ws, bs):
        flat += [w, b]
    return pl.pallas_call(
        _enc_vq_body,
        grid=(nb,),
        in_specs=in_specs,
        out_specs=[pl.BlockSpec((_BB1, ED), lambda i: (i, 0)),
                   pl.BlockSpec((_BB1, 1), lambda i: (i, 0))],
        out_shape=[jax.ShapeDtypeStruct((B, ED), jnp.float32),
                   jax.ShapeDtypeStruct((B, 1), jnp.int32)],
        scratch_shapes=[pltpu.VMEM((1, K), jnp.float32)],
    )(e, *flat, cb)


def _dec_body(zqr_ref, ze_ref, idx_ref, tgt_ref,
              w0, b0, w1, b1, w2, b2, w3, b3,
              pred_ref, zq_ref, com_ref, perp_ref, rec_ref, tot_ref,
              counts_ref, sq_ref, r_ref):
    i = pl.program_id(0)
    nb = pl.num_programs(0)

    zqr = zqr_ref[...]
    ze = ze_ref[...]
    zq = ze + (zqr - ze)
    zq_ref[...] = zq

    h = jnp.maximum(_mm(zq, w0[...]) + b0[...], 0.0)
    h = jnp.maximum(_mm(h, w1[...]) + b1[...], 0.0)
    h = jnp.maximum(_mm(h, w2[...]) + b2[...], 0.0)
    pred = _mm(h, w3[...]) + b3[...]
    pred_ref[...] = pred

    sq = jnp.sum((zqr - ze) ** 2)
    rc = jnp.sum((pred - tgt_ref[...]) ** 2)
    ii = lax.broadcasted_iota(jnp.int32, (_BB2, K), 1)
    oh = (idx_ref[...] == ii).astype(jnp.float32)
    cnt = jnp.sum(oh, axis=0)[None, :]

    @pl.when(i == 0)
    def _():
        sq_ref[0] = 0.0
        r_ref[0] = 0.0
        counts_ref[...] = jnp.zeros_like(counts_ref)

    sq_ref[0] += sq
    r_ref[0] += rc
    counts_ref[...] += cnt

    @pl.when(i == nb - 1)
    def _():
        a = sq_ref[0] * (1.0 / (B * ED))
        com = a + BETA * a
        rec = r_ref[0] * (1.0 / (B * OUT))
        p = counts_ref[...] * (1.0 / B)
        ent = -jnp.sum(p * jnp.log(p + 1e-10))
        com_ref[...] = jnp.broadcast_to(com, (1, 1))
        perp_ref[...] = jnp.broadcast_to(jnp.exp(ent), (1, 1))
        rec_ref[...] = jnp.broadcast_to(rec, (1, 1))
        tot_ref[...] = jnp.broadcast_to(rec + com, (1, 1))


def _decode(zq_raw, z_e, idx2, targets, ws, bs):
    nb = B // _BB2
    full2 = lambda shape: pl.BlockSpec(shape, lambda i: (0, 0))
    in_specs = [pl.BlockSpec((_BB2, ED), lambda i: (i, 0)),
                pl.BlockSpec((_BB2, ED), lambda i: (i, 0)),
                pl.BlockSpec((_BB2, 1), lambda i: (i, 0)),
                pl.BlockSpec((_BB2, OUT), lambda i: (i, 0))]
    flat = []
    for w, b in zip(ws, bs):
        in_specs.append(full2(w.shape))
        in_specs.append(full2(b.shape))
        flat += [w, b]
    scal = lambda: pl.BlockSpec((1, 1), lambda i: (0, 0))
    return pl.pallas_call(
        _dec_body,
        grid=(nb,),
        in_specs=in_specs,
        out_specs=[pl.BlockSpec((_BB2, OUT), lambda i: (i, 0)),
                   pl.BlockSpec((_BB2, ED), lambda i: (i, 0)),
                   scal(), scal(), scal(), scal()],
        out_shape=[jax.ShapeDtypeStruct((B, OUT), jnp.float32),
                   jax.ShapeDtypeStruct((B, ED), jnp.float32),
                   jax.ShapeDtypeStruct((1, 1), jnp.float32),
                   jax.ShapeDtypeStruct((1, 1), jnp.float32),
                   jax.ShapeDtypeStruct((1, 1), jnp.float32),
                   jax.ShapeDtypeStruct((1, 1), jnp.float32)],
        scratch_shapes=[pltpu.VMEM((1, K), jnp.float32),
                        pltpu.SMEM((1,), jnp.float32),
                        pltpu.SMEM((1,), jnp.float32)],
    )(zq_raw, z_e, idx2, targets, *flat)


def kernel(tokens, targets, tok_embed,
           enc_W0, enc_b0, enc_W1, enc_b1, enc_W2, enc_b2, enc_W3, enc_b3,
           codebook,
           dec_W0, dec_b0, dec_W1, dec_b1, dec_W2, dec_b2, dec_W3, dec_b3):
    enc_ws = [enc_W0, enc_W1, enc_W2, enc_W3]
    enc_bs = [b.reshape(1, -1) for b in (enc_b0, enc_b1, enc_b2, enc_b3)]
    dec_ws = [dec_W0, dec_W1, dec_W2, dec_W3]
    dec_bs = [b.reshape(1, -1) for b in (dec_b0, dec_b1, dec_b2, dec_b3)]

    e = _sc_gather_rows(tok_embed, tokens.reshape(-1)).reshape(B, NT * TOK_EMB)
    z_e, idx2 = _enc_vq(e, enc_ws, enc_bs, codebook)
    zq_raw = _sc_gather_rows(codebook, idx2.reshape(-1))
    pred, zq, com, perp, rec, tot = _decode(zq_raw, z_e, idx2, targets,
                                            dec_ws, dec_bs)
    indices = idx2.reshape(B)
    return (pred, com.reshape(()), perp.reshape(()), indices, z_e, zq,
            rec.reshape(()), tot.reshape(()))
